# Initial kernel scaffold; baseline (speedup 1.0000x reference)
import jax
import jax.numpy as jnp
from jax.experimental import pallas as pl


def kernel(x, W):
    raise NotImplementedError("write your pallas kernel here")


# speedup vs baseline: 2.2210x; 2.2210x over previous
import functools

import jax
import jax.numpy as jnp
from jax.experimental import pallas as pl

D_MODEL = 768
NUM_EXPERTS = 64
TOKEN_BLOCK = 1024


def _router_body(x_ref, w_ref, g_ref, i_ref):
    x = x_ref[...]
    w = w_ref[...]
    logits = jax.lax.dot_general(
        x, w, (((1,), (1,)), ((), ())),
        preferred_element_type=jnp.float32)
    i1 = jnp.argmax(logits, axis=-1)
    m1 = jnp.max(logits, axis=-1, keepdims=True)
    iota = jax.lax.broadcasted_iota(jnp.int32, logits.shape, 1)
    masked = jnp.where(iota == i1[:, None], -jnp.inf, logits)
    i2 = jnp.argmax(masked, axis=-1)
    m2 = jnp.max(masked, axis=-1, keepdims=True)
    g1 = 1.0 / (1.0 + jnp.exp(m2 - m1))
    g_ref[...] = jnp.concatenate([g1, 1.0 - g1], axis=1)
    i_ref[...] = jnp.stack([i1, i2], axis=1).astype(jnp.int32)


@functools.partial(jax.jit, static_argnames=())
def kernel(x, W):
    B, S, D = x.shape
    N = B * S
    xf = x.reshape(N, D)
    T = TOKEN_BLOCK
    grid = (N // T,)
    gates, indices = pl.pallas_call(
        _router_body,
        grid=grid,
        in_specs=[
            pl.BlockSpec((T, D), lambda i: (i, 0)),
            pl.BlockSpec((NUM_EXPERTS, D), lambda i: (0, 0)),
        ],
        out_specs=[
            pl.BlockSpec((T, 2), lambda i: (i, 0)),
            pl.BlockSpec((T, 2), lambda i: (i, 0)),
        ],
        out_shape=[
            jax.ShapeDtypeStruct((N, 2), jnp.float32),
            jax.ShapeDtypeStruct((N, 2), jnp.int32),
        ],
    )(xf, W)
    return gates.reshape(B, S, 2), indices.reshape(B, S, 2)


# speedup vs baseline: 2.5189x; 1.1341x over previous
import functools

import jax
import jax.numpy as jnp
from jax.experimental import pallas as pl

D_MODEL = 768
NUM_EXPERTS = 64
TOKEN_BLOCK = 2048


def _router_body(x_ref, w_ref, g_ref, i_ref):
    x = x_ref[...]
    w = w_ref[...]
    logits = jax.lax.dot_general(
        x, w, (((1,), (1,)), ((), ())),
        preferred_element_type=jnp.float32)
    i1 = jnp.argmax(logits, axis=-1)
    m1 = jnp.max(logits, axis=-1, keepdims=True)
    iota = jax.lax.broadcasted_iota(jnp.int32, logits.shape, 1)
    masked = jnp.where(iota == i1[:, None], -jnp.inf, logits)
    i2 = jnp.argmax(masked, axis=-1)
    m2 = jnp.max(masked, axis=-1, keepdims=True)
    g1 = 1.0 / (1.0 + jnp.exp(m2 - m1))
    g_ref[...] = jnp.concatenate([g1, 1.0 - g1], axis=1)
    i_ref[...] = jnp.stack([i1, i2], axis=1).astype(jnp.int32)


@functools.partial(jax.jit, static_argnames=())
def kernel(x, W):
    B, S, D = x.shape
    N = B * S
    xf = x.reshape(N, D)
    T = TOKEN_BLOCK
    grid = (N // T,)
    gates, indices = pl.pallas_call(
        _router_body,
        grid=grid,
        in_specs=[
            pl.BlockSpec((T, D), lambda i: (i, 0)),
            pl.BlockSpec((NUM_EXPERTS, D), lambda i: (0, 0)),
        ],
        out_specs=[
            pl.BlockSpec((T, 2), lambda i: (i, 0)),
            pl.BlockSpec((T, 2), lambda i: (i, 0)),
        ],
        out_shape=[
            jax.ShapeDtypeStruct((N, 2), jnp.float32),
            jax.ShapeDtypeStruct((N, 2), jnp.int32),
        ],
    )(xf, W)
    return gates.reshape(B, S, 2), indices.reshape(B, S, 2)


# speedup vs baseline: 2.6512x; 1.0525x over previous
import functools

import jax
import jax.numpy as jnp
from jax.experimental import pallas as pl

D_MODEL = 768
NUM_EXPERTS = 64
TOKEN_BLOCK = 4096


def _router_body(x_ref, w_ref, g_ref, i_ref):
    x = x_ref[...]
    w = w_ref[...]
    logits = jax.lax.dot_general(
        x, w, (((1,), (1,)), ((), ())),
        preferred_element_type=jnp.float32)
    i1 = jnp.argmax(logits, axis=-1)
    m1 = jnp.max(logits, axis=-1, keepdims=True)
    iota = jax.lax.broadcasted_iota(jnp.int32, logits.shape, 1)
    masked = jnp.where(iota == i1[:, None], -jnp.inf, logits)
    i2 = jnp.argmax(masked, axis=-1)
    m2 = jnp.max(masked, axis=-1, keepdims=True)
    g1 = 1.0 / (1.0 + jnp.exp(m2 - m1))
    g_ref[...] = jnp.concatenate([g1, 1.0 - g1], axis=1)
    i_ref[...] = jnp.stack([i1, i2], axis=1).astype(jnp.int32)


@functools.partial(jax.jit, static_argnames=())
def kernel(x, W):
    B, S, D = x.shape
    N = B * S
    xf = x.reshape(N, D)
    T = TOKEN_BLOCK
    grid = (N // T,)
    gates, indices = pl.pallas_call(
        _router_body,
        grid=grid,
        in_specs=[
            pl.BlockSpec((T, D), lambda i: (i, 0)),
            pl.BlockSpec((NUM_EXPERTS, D), lambda i: (0, 0)),
        ],
        out_specs=[
            pl.BlockSpec((T, 2), lambda i: (i, 0)),
            pl.BlockSpec((T, 2), lambda i: (i, 0)),
        ],
        out_shape=[
            jax.ShapeDtypeStruct((N, 2), jnp.float32),
            jax.ShapeDtypeStruct((N, 2), jnp.int32),
        ],
    )(xf, W)
    return gates.reshape(B, S, 2), indices.reshape(B, S, 2)


# speedup vs baseline: 4.4112x; 1.6639x over previous
import jax
import jax.numpy as jnp
from jax.experimental import pallas as pl

D_MODEL = 768
NUM_EXPERTS = 64
TOKEN_BLOCK = 2048
N_STREAMS = 4


def _router_body(*refs):
    w = refs[N_STREAMS][...]
    g_ref = refs[N_STREAMS + 1]
    i_ref = refs[N_STREAMS + 2]
    T = TOKEN_BLOCK
    for s in range(N_STREAMS):
        x = refs[s][...]
        logits = jax.lax.dot_general(
            x, w, (((1,), (1,)), ((), ())),
            preferred_element_type=jnp.float32)
        i1 = jnp.argmax(logits, axis=-1)
        m1 = jnp.max(logits, axis=-1)
        iota = jax.lax.broadcasted_iota(jnp.int32, logits.shape, 1)
        masked = jnp.where(iota == i1[:, None], -jnp.inf, logits)
        i2 = jnp.argmax(masked, axis=-1)
        m2 = jnp.max(masked, axis=-1)
        g1 = 1.0 / (1.0 + jnp.exp(m2 - m1))
        g_ref[:, s * T:(s + 1) * T] = jnp.stack([g1, 1.0 - g1], axis=0)
        i_ref[:, s * T:(s + 1) * T] = jnp.stack([i1, i2], axis=0).astype(jnp.int32)


def kernel(x, W):
    B, S, D = x.shape
    N = B * S
    xf = x.reshape(N, D)
    T = TOKEN_BLOCK
    ns = N_STREAMS
    grid = (N // (T * ns),)

    def x_spec(s):
        return pl.BlockSpec((T, D), lambda i, s=s: (ns * i + s, 0))

    gates_t, indices_t = pl.pallas_call(
        _router_body,
        grid=grid,
        in_specs=[x_spec(s) for s in range(ns)]
        + [pl.BlockSpec((NUM_EXPERTS, D), lambda i: (0, 0))],
        out_specs=[
            pl.BlockSpec((2, ns * T), lambda i: (0, i)),
            pl.BlockSpec((2, ns * T), lambda i: (0, i)),
        ],
        out_shape=[
            jax.ShapeDtypeStruct((2, N), jnp.float32),
            jax.ShapeDtypeStruct((2, N), jnp.int32),
        ],
    )(*([xf] * ns), W)
    gates = gates_t.T.reshape(B, S, 2)
    indices = indices_t.T.reshape(B, S, 2)
    return gates, indices


# speedup vs baseline: 4.4498x; 1.0088x over previous
import jax
import jax.numpy as jnp
from jax.experimental import pallas as pl

D_MODEL = 768
NUM_EXPERTS = 64
TOKEN_BLOCK = 1024
N_STREAMS = 8


def _router_body(*refs):
    w = refs[N_STREAMS][...]
    g_ref = refs[N_STREAMS + 1]
    i_ref = refs[N_STREAMS + 2]
    T = TOKEN_BLOCK
    for s in range(N_STREAMS):
        x = refs[s][...]
        logits = jax.lax.dot_general(
            x, w, (((1,), (1,)), ((), ())),
            preferred_element_type=jnp.float32)
        i1 = jnp.argmax(logits, axis=-1)
        m1 = jnp.max(logits, axis=-1)
        iota = jax.lax.broadcasted_iota(jnp.int32, logits.shape, 1)
        masked = jnp.where(iota == i1[:, None], -jnp.inf, logits)
        i2 = jnp.argmax(masked, axis=-1)
        m2 = jnp.max(masked, axis=-1)
        g1 = 1.0 / (1.0 + jnp.exp(m2 - m1))
        g_ref[:, s * T:(s + 1) * T] = jnp.stack([g1, 1.0 - g1], axis=0)
        i_ref[:, s * T:(s + 1) * T] = jnp.stack([i1, i2], axis=0).astype(jnp.int32)


def kernel(x, W):
    B, S, D = x.shape
    N = B * S
    xf = x.reshape(N, D)
    T = TOKEN_BLOCK
    ns = N_STREAMS
    grid = (N // (T * ns),)

    def x_spec(s):
        return pl.BlockSpec((T, D), lambda i, s=s: (ns * i + s, 0))

    gates_t, indices_t = pl.pallas_call(
        _router_body,
        grid=grid,
        in_specs=[x_spec(s) for s in range(ns)]
        + [pl.BlockSpec((NUM_EXPERTS, D), lambda i: (0, 0))],
        out_specs=[
            pl.BlockSpec((2, ns * T), lambda i: (0, i)),
            pl.BlockSpec((2, ns * T), lambda i: (0, i)),
        ],
        out_shape=[
            jax.ShapeDtypeStruct((2, N), jnp.float32),
            jax.ShapeDtypeStruct((2, N), jnp.int32),
        ],
    )(*([xf] * ns), W)
    gates = gates_t.T.reshape(B, S, 2)
    indices = indices_t.T.reshape(B, S, 2)
    return gates, indices


# speedup vs baseline: 4.4713x; 1.0048x over previous
import jax
import jax.numpy as jnp
from jax.experimental import pallas as pl

D_MODEL = 768
NUM_EXPERTS = 64
TOKEN_BLOCK = 512
N_STREAMS = 16


def _router_body(*refs):
    w = refs[N_STREAMS][...]
    g_ref = refs[N_STREAMS + 1]
    i_ref = refs[N_STREAMS + 2]
    T = TOKEN_BLOCK
    for s in range(N_STREAMS):
        x = refs[s][...]
        logits = jax.lax.dot_general(
            x, w, (((1,), (1,)), ((), ())),
            preferred_element_type=jnp.float32)
        i1 = jnp.argmax(logits, axis=-1)
        m1 = jnp.max(logits, axis=-1)
        iota = jax.lax.broadcasted_iota(jnp.int32, logits.shape, 1)
        masked = jnp.where(iota == i1[:, None], -jnp.inf, logits)
        i2 = jnp.argmax(masked, axis=-1)
        m2 = jnp.max(masked, axis=-1)
        g1 = 1.0 / (1.0 + jnp.exp(m2 - m1))
        g_ref[:, s * T:(s + 1) * T] = jnp.stack([g1, 1.0 - g1], axis=0)
        i_ref[:, s * T:(s + 1) * T] = jnp.stack([i1, i2], axis=0).astype(jnp.int32)


def kernel(x, W):
    B, S, D = x.shape
    N = B * S
    xf = x.reshape(N, D)
    T = TOKEN_BLOCK
    ns = N_STREAMS
    grid = (N // (T * ns),)

    def x_spec(s):
        return pl.BlockSpec((T, D), lambda i, s=s: (ns * i + s, 0))

    gates_t, indices_t = pl.pallas_call(
        _router_body,
        grid=grid,
        in_specs=[x_spec(s) for s in range(ns)]
        + [pl.BlockSpec((NUM_EXPERTS, D), lambda i: (0, 0))],
        out_specs=[
            pl.BlockSpec((2, ns * T), lambda i: (0, i)),
            pl.BlockSpec((2, ns * T), lambda i: (0, i)),
        ],
        out_shape=[
            jax.ShapeDtypeStruct((2, N), jnp.float32),
            jax.ShapeDtypeStruct((2, N), jnp.int32),
        ],
    )(*([xf] * ns), W)
    gates = gates_t.T.reshape(B, S, 2)
    indices = indices_t.T.reshape(B, S, 2)
    return gates, indices
